# baseline (speedup 1.0000x reference)
import functools
import math

import jax
import jax.numpy as jnp
from jax import lax
from jax.experimental import pallas as pl
from jax.experimental.pallas import tpu as pltpu
from jax.experimental.pallas import tpu_sc as plsc

_NC = 2
_NS = 16
_NW = _NC * _NS
_CHUNK = 128
_LANES = 16


@functools.partial(jax.jit, static_argnums=())
def _lookup(x_flat, table):
    B = x_flat.shape[0]
    V, D = table.shape
    b_per_w = B // _NW
    n_chunks = b_per_w // _CHUNK
    scale = math.sqrt(D)
    mesh = plsc.VectorSubcoreMesh(core_axis_name="c", subcore_axis_name="s")

    idx2d = x_flat.reshape(_NW * n_chunks, _CHUNK)

    @functools.partial(
        pl.kernel,
        out_type=jax.ShapeDtypeStruct((B, D), jnp.float32),
        mesh=mesh,
        compiler_params=pltpu.CompilerParams(use_tc_tiling_on_sc=False),
        scratch_types=[
            pltpu.VMEM((n_chunks, _CHUNK), jnp.int32),
            pltpu.VMEM((_CHUNK, D), jnp.float32),
            pltpu.SemaphoreType.DMA,
        ],
    )
    def look(idx_hbm, table_hbm, out_hbm, idx_v, rows_v, sem):
        wid = lax.axis_index("s") * _NC + lax.axis_index("c")
        base = wid * b_per_w
        pltpu.sync_copy(idx_hbm.at[pl.ds(wid * n_chunks, n_chunks)], idx_v)

        @pl.loop(0, n_chunks)
        def chunk_body(j):
            cp = pltpu.async_copy(table_hbm.at[idx_v.at[j]], rows_v, sem)
            cp.wait()

            @pl.loop(0, _CHUNK)
            def scale_body(i):
                for t in range(D // _LANES):
                    sl = pl.ds(t * _LANES, _LANES)
                    rows_v[i, sl] = rows_v[i, sl] * scale

            pltpu.sync_copy(rows_v, out_hbm.at[pl.ds(base + j * _CHUNK, _CHUNK)])

    return look(idx2d, table)


def kernel(x, table):
    B = x.shape[0] * x.shape[1]
    out = _lookup(x.reshape(-1).astype(jnp.int32), table)
    return out.reshape(x.shape[0], x.shape[1], table.shape[1])


# speedup vs baseline: 1.2118x; 1.2118x over previous
import functools
import math

import jax
import jax.numpy as jnp
from jax import lax
from jax.experimental import pallas as pl
from jax.experimental.pallas import tpu as pltpu
from jax.experimental.pallas import tpu_sc as plsc

_NC = 2
_NS = 16
_NW = _NC * _NS
_CHUNK = 128
_LANES = 16
_RING = 4
_AHEAD = 2


@jax.jit
def _lookup(x_flat, table):
    B = x_flat.shape[0]
    V, D = table.shape
    b_per_w = B // _NW
    n_chunks = b_per_w // _CHUNK
    scale = float(math.sqrt(D))
    mesh = plsc.VectorSubcoreMesh(core_axis_name="c", subcore_axis_name="s")

    idx2d = x_flat.reshape(_NW * n_chunks, _CHUNK)

    @functools.partial(
        pl.kernel,
        out_type=jax.ShapeDtypeStruct((B, D), jnp.float32),
        mesh=mesh,
        compiler_params=pltpu.CompilerParams(use_tc_tiling_on_sc=False),
        scratch_types=[
            pltpu.VMEM((n_chunks, _CHUNK), jnp.int32),
            pltpu.VMEM((_RING, _CHUNK, D), jnp.float32),
            pltpu.SemaphoreType.DMA((_RING,)),
            pltpu.SemaphoreType.DMA((_RING,)),
        ],
    )
    def look(idx_hbm, table_hbm, out_hbm, idx_v, bufs, sem_g, sem_s):
        wid = lax.axis_index("s") * _NC + lax.axis_index("c")
        base = wid * b_per_w
        pltpu.sync_copy(idx_hbm.at[pl.ds(wid * n_chunks, n_chunks)], idx_v)

        def gather(j, b):
            return pltpu.make_async_copy(
                table_hbm.at[idx_v.at[j]], bufs.at[b], sem_g.at[b])

        def scatter(j, b):
            return pltpu.make_async_copy(
                bufs.at[b], out_hbm.at[pl.ds(base + j * _CHUNK, _CHUNK)],
                sem_s.at[b])

        for b in range(_AHEAD):
            gather(b, b).start()

        @pl.loop(0, n_chunks, step=_RING)
        def outer(j0):
            for b in range(_RING):
                j = j0 + b
                bb = (b + _AHEAD) % _RING
                jg = j + _AHEAD

                @pl.when(jg < n_chunks)
                def _():
                    @pl.when(jg >= _RING)
                    def _():
                        scatter(jg - _RING, bb).wait()

                    gather(jg, bb).start()

                gather(j, b).wait()

                @pl.loop(0, _CHUNK, unroll=8)
                def scale_body(i):
                    for t in range(D // _LANES):
                        sl = pl.ds(t * _LANES, _LANES)
                        bufs[b, i, sl] = bufs[b, i, sl] * scale

                scatter(j, b).start()

        for b in range(_RING):
            scatter(n_chunks - _RING + b, b).wait()

    return look(idx2d, table)


def kernel(x, table):
    out = _lookup(x.reshape(-1).astype(jnp.int32), table)
    return out.reshape(x.shape[0], x.shape[1], table.shape[1])
